# baseline (speedup 1.0000x reference)
import functools

import jax
import jax.numpy as jnp
from jax import lax
from jax.experimental import pallas as pl
from jax.experimental.pallas import tpu as pltpu
from jax.experimental.pallas import tpu_sc as plsc

N = 10000
E = 320000
D_IN = 128
D_LAT = 64

NC = 2
NS = 16
L = 16
NW = NC * NS
K = 80
EPW = E // NW
NCHUNK = EPW // K
CB = 25
NB = NCHUNK // CB
NZ = 10
ZROWS = N // NZ
ZB = 40


def _sc_aggregate(x, row2, col2, val2, zrows_hbm):
  mesh = plsc.VectorSubcoreMesh(core_axis_name="c", subcore_axis_name="s",
                                num_cores=NC, num_subcores=NS)

  @functools.partial(
      pl.kernel,
      out_type=jax.ShapeDtypeStruct((NC * N, D_IN), jnp.float32),
      mesh=mesh,
      scratch_types=[
          pltpu.VMEM((CB, K), jnp.int32),
          pltpu.VMEM((CB, K), jnp.int32),
          pltpu.VMEM((CB, K), jnp.float32),
          pltpu.VMEM((K, D_IN), jnp.float32),
          pltpu.VMEM_SHARED((N, D_IN), jnp.float32),
          pltpu.SemaphoreType.DMA,
      ],
  )
  def agg_kernel(x_hbm, row_hbm, col_hbm, val_hbm, z_hbm, out_hbm,
                 rowv, colv, valv, rows_v, acc, sem):
    c = lax.axis_index("c")
    s = lax.axis_index("s")
    wid = s * NC + c

    @pl.when(s < NZ)
    def _zero():
      pltpu.sync_copy(z_hbm, acc.at[pl.ds(s * ZROWS, ZROWS)])
    plsc.subcore_barrier()

    def block_body(b, _):
      pltpu.sync_copy(row_hbm.at[wid, b], rowv)
      pltpu.sync_copy(col_hbm.at[wid, b], colv)
      pltpu.sync_copy(val_hbm.at[wid, b], valv)

      def chunk_body(j, _):
        pltpu.async_copy(x_hbm.at[colv.at[j]], rows_v, sem).wait()

        def grp_body(g, _):
          vv = valv[j, pl.ds(g * L, L)]
          for l in range(L):
            v = vv[l]
            e = g * L + l
            for t in range(D_IN // L):
              sl = pl.ds(t * L, L)
              rows_v[e, sl] = rows_v[e, sl] * v
          return 0
        lax.fori_loop(0, K // L, grp_body, 0)

        pltpu.sync_copy(rows_v, acc.at[rowv.at[j]], add=True)
        return 0

      lax.fori_loop(0, CB, chunk_body, 0)
      return 0

    lax.fori_loop(0, NB, block_body, 0)
    plsc.subcore_barrier()

    @pl.when(s < NZ)
    def _dump():
      pltpu.sync_copy(acc.at[pl.ds(s * ZROWS, ZROWS)],
                      out_hbm.at[pl.ds(c * N + s * ZROWS, ZROWS)])

  return agg_kernel(x, row2, col2, val2, zrows_hbm)


def _tc_tail(partials, wt, bias):
  BLK = 1000
  grid = (N // BLK,)

  def tail_kernel(p0_ref, p1_ref, wt_ref, b_ref, mu_ref, lv_ref):
    agg = p0_ref[...] + p1_ref[...]
    y = jnp.dot(agg, wt_ref[...], preferred_element_type=jnp.float32)
    y = y + b_ref[...]
    mu_ref[...] = y[:, :D_LAT]
    lv_ref[...] = jnp.clip(y[:, D_LAT:], -10.0, 3.0)

  return pl.pallas_call(
      tail_kernel,
      grid=grid,
      in_specs=[
          pl.BlockSpec((BLK, D_IN), lambda i: (i, 0)),
          pl.BlockSpec((BLK, D_IN), lambda i: (i + N // BLK, 0)),
          pl.BlockSpec((D_IN, 2 * D_LAT), lambda i: (0, 0)),
          pl.BlockSpec((1, 2 * D_LAT), lambda i: (0, 0)),
      ],
      out_specs=[
          pl.BlockSpec((BLK, D_LAT), lambda i: (i, 0)),
          pl.BlockSpec((BLK, D_LAT), lambda i: (i, 0)),
      ],
      out_shape=[
          jax.ShapeDtypeStruct((N, D_LAT), jnp.float32),
          jax.ShapeDtypeStruct((N, D_LAT), jnp.float32),
      ],
  )(partials, partials, wt, bias)


@jax.jit
def kernel(x, adj_indices, adj_values, W_mu, b_mu, W_lv, b_lv):
  row2 = adj_indices[0].astype(jnp.int32).reshape(NW, NB, CB, K)
  col2 = adj_indices[1].astype(jnp.int32).reshape(NW, NB, CB, K)
  val2 = adj_values.reshape(NW, NB, CB, K)

  zrows = jnp.zeros((ZROWS, D_IN), jnp.float32)
  partials = _sc_aggregate(x, row2, col2, val2, zrows)

  wt = jnp.concatenate([W_mu, W_lv], axis=0).T
  bias = jnp.concatenate([b_mu, b_lv]).reshape(1, 2 * D_LAT)
  mu, logvar = _tc_tail(partials, wt, bias)
  return (mu, logvar)


# speedup vs baseline: 1.1613x; 1.1613x over previous
import functools

import jax
import jax.numpy as jnp
from jax import lax
from jax.experimental import pallas as pl
from jax.experimental.pallas import tpu as pltpu
from jax.experimental.pallas import tpu_sc as plsc

N = 10000
E = 320000
D_IN = 128
D_LAT = 64

NC = 2
NS = 16
L = 16
NW = NC * NS
K = 80
EPW = E // NW
NCHUNK = EPW // K
CB = 25
NB = NCHUNK // CB
NZ = 10
ZROWS = N // NZ
ZB = 40


def _sc_aggregate(x, row2, col2, val2, zrows_hbm):
  mesh = plsc.VectorSubcoreMesh(core_axis_name="c", subcore_axis_name="s",
                                num_cores=NC, num_subcores=NS)

  @functools.partial(
      pl.kernel,
      out_type=jax.ShapeDtypeStruct((NC * N, D_IN), jnp.float32),
      mesh=mesh,
      scratch_types=[
          pltpu.VMEM((CB, K), jnp.int32),
          pltpu.VMEM((CB, K), jnp.int32),
          pltpu.VMEM((CB, K), jnp.float32),
          pltpu.VMEM((K, D_IN), jnp.float32),
          pltpu.VMEM((K, D_IN), jnp.float32),
          pltpu.VMEM_SHARED((N, D_IN), jnp.float32),
          pltpu.SemaphoreType.DMA,
          pltpu.SemaphoreType.DMA,
      ],
  )
  def agg_kernel(x_hbm, row_hbm, col_hbm, val_hbm, z_hbm, out_hbm,
                 rowv, colv, valv, rows_a, rows_b, acc, ga, gb):
    c = lax.axis_index("c")
    s = lax.axis_index("s")
    wid = s * NC + c

    @pl.when(s < NZ)
    def _zero():
      pltpu.sync_copy(z_hbm, acc.at[pl.ds(s * ZROWS, ZROWS)])
    plsc.subcore_barrier()

    def block_body(b, _):
      pltpu.sync_copy(row_hbm.at[wid, b], rowv)
      pltpu.sync_copy(col_hbm.at[wid, b], colv)
      pltpu.sync_copy(val_hbm.at[wid, b], valv)

      def scale(buf, j):
        def grp_body(g, _):
          vv = valv[j, pl.ds(g * L, L)]
          for l in range(L):
            v = vv[l]
            e = g * L + l
            for t in range(D_IN // L):
              sl = pl.ds(t * L, L)
              buf[e, sl] = buf[e, sl] * v
          return 0
        lax.fori_loop(0, K // L, grp_body, 0)

      def pair_body(p, _):
        j0 = 2 * p
        j1 = 2 * p + 1
        cp_a = pltpu.async_copy(x_hbm.at[colv.at[j0]], rows_a, ga)
        cp_b = pltpu.async_copy(x_hbm.at[colv.at[j1]], rows_b, gb)
        cp_a.wait()
        scale(rows_a, j0)
        pltpu.sync_copy(rows_a, acc.at[rowv.at[j0]], add=True)
        cp_b.wait()
        scale(rows_b, j1)
        pltpu.sync_copy(rows_b, acc.at[rowv.at[j1]], add=True)
        return 0

      lax.fori_loop(0, CB // 2, pair_body, 0)

      jl = CB - 1
      pltpu.async_copy(x_hbm.at[colv.at[jl]], rows_a, ga).wait()
      scale(rows_a, jl)
      pltpu.sync_copy(rows_a, acc.at[rowv.at[jl]], add=True)
      return 0

    lax.fori_loop(0, NB, block_body, 0)
    plsc.subcore_barrier()

    @pl.when(s < NZ)
    def _dump():
      pltpu.sync_copy(acc.at[pl.ds(s * ZROWS, ZROWS)],
                      out_hbm.at[pl.ds(c * N + s * ZROWS, ZROWS)])

  return agg_kernel(x, row2, col2, val2, zrows_hbm)


def _tc_tail(partials, wt, bias):
  BLK = 1000
  grid = (N // BLK,)

  def tail_kernel(p0_ref, p1_ref, wt_ref, b_ref, mu_ref, lv_ref):
    agg = p0_ref[...] + p1_ref[...]
    y = jnp.dot(agg, wt_ref[...], preferred_element_type=jnp.float32)
    y = y + b_ref[...]
    mu_ref[...] = y[:, :D_LAT]
    lv_ref[...] = jnp.clip(y[:, D_LAT:], -10.0, 3.0)

  return pl.pallas_call(
      tail_kernel,
      grid=grid,
      in_specs=[
          pl.BlockSpec((BLK, D_IN), lambda i: (i, 0)),
          pl.BlockSpec((BLK, D_IN), lambda i: (i + N // BLK, 0)),
          pl.BlockSpec((D_IN, 2 * D_LAT), lambda i: (0, 0)),
          pl.BlockSpec((1, 2 * D_LAT), lambda i: (0, 0)),
      ],
      out_specs=[
          pl.BlockSpec((BLK, D_LAT), lambda i: (i, 0)),
          pl.BlockSpec((BLK, D_LAT), lambda i: (i, 0)),
      ],
      out_shape=[
          jax.ShapeDtypeStruct((N, D_LAT), jnp.float32),
          jax.ShapeDtypeStruct((N, D_LAT), jnp.float32),
      ],
  )(partials, partials, wt, bias)


@jax.jit
def kernel(x, adj_indices, adj_values, W_mu, b_mu, W_lv, b_lv):
  row2 = adj_indices[0].astype(jnp.int32).reshape(NW, NB, CB, K)
  col2 = adj_indices[1].astype(jnp.int32).reshape(NW, NB, CB, K)
  val2 = adj_values.reshape(NW, NB, CB, K)

  zrows = jnp.zeros((ZROWS, D_IN), jnp.float32)
  partials = _sc_aggregate(x, row2, col2, val2, zrows)

  wt = jnp.concatenate([W_mu, W_lv], axis=0).T
  bias = jnp.concatenate([b_mu, b_lv]).reshape(1, 2 * D_LAT)
  mu, logvar = _tc_tail(partials, wt, bias)
  return (mu, logvar)


# speedup vs baseline: 1.2560x; 1.0816x over previous
import functools

import jax
import jax.numpy as jnp
from jax import lax
from jax.experimental import pallas as pl
from jax.experimental.pallas import tpu as pltpu
from jax.experimental.pallas import tpu_sc as plsc

N = 10000
E = 320000
D_IN = 128
D_LAT = 64

NC = 2
NS = 16
L = 16
NW = NC * NS
K = 80
EPW = E // NW
NCHUNK = EPW // K
CB = 25
NB = NCHUNK // CB
NZ = 10
ZROWS = N // NZ
ZB = 40


def _sc_aggregate(x, row2, col2, val2, zrows_hbm):
  mesh = plsc.VectorSubcoreMesh(core_axis_name="c", subcore_axis_name="s",
                                num_cores=NC, num_subcores=NS)

  @functools.partial(
      pl.kernel,
      out_type=jax.ShapeDtypeStruct((NC * N, D_IN), jnp.float32),
      mesh=mesh,
      scratch_types=[
          pltpu.VMEM((CB, K), jnp.int32),
          pltpu.VMEM((CB, K), jnp.int32),
          pltpu.VMEM((CB, K), jnp.float32),
          pltpu.VMEM((K, D_IN), jnp.float32),
          pltpu.VMEM((K, D_IN), jnp.float32),
          pltpu.VMEM_SHARED((N, D_IN), jnp.float32),
          pltpu.SemaphoreType.DMA,
          pltpu.SemaphoreType.DMA,
      ],
  )
  def agg_kernel(x_hbm, row_hbm, col_hbm, val_hbm, z_hbm, out_hbm,
                 rowv, colv, valv, rows_a, rows_b, acc, ga, gb):
    c = lax.axis_index("c")
    s = lax.axis_index("s")
    wid = s * NC + c

    @pl.when(s < NZ)
    def _zero():
      pltpu.sync_copy(z_hbm, acc.at[pl.ds(s * ZROWS, ZROWS)])
    plsc.subcore_barrier()

    def block_body(b, _):
      pltpu.sync_copy(row_hbm.at[wid, b], rowv)
      pltpu.sync_copy(col_hbm.at[wid, b], colv)
      pltpu.sync_copy(val_hbm.at[wid, b], valv)

      def scale(buf, j):
        def grp_body(g, _):
          vv = valv[j, pl.ds(g * L, L)]
          for l in range(L):
            v = vv[l]
            e = g * L + l
            for t in range(D_IN // L):
              sl = pl.ds(t * L, L)
              buf[e, sl] = buf[e, sl] * v
          return 0
        lax.fori_loop(0, K // L, grp_body, 0)

      def pair_body(p, _):
        j0 = 2 * p
        j1 = 2 * p + 1
        cp_a = pltpu.async_copy(x_hbm.at[colv.at[j0]], rows_a, ga)
        cp_b = pltpu.async_copy(x_hbm.at[colv.at[j1]], rows_b, gb)
        cp_a.wait()
        scale(rows_a, j0)
        sc_a = pltpu.async_copy(rows_a, acc.at[rowv.at[j0]], ga, add=True)
        cp_b.wait()
        scale(rows_b, j1)
        pltpu.sync_copy(rows_b, acc.at[rowv.at[j1]], add=True)
        sc_a.wait()
        return 0

      lax.fori_loop(0, CB // 2, pair_body, 0)

      jl = CB - 1
      pltpu.async_copy(x_hbm.at[colv.at[jl]], rows_a, ga).wait()
      scale(rows_a, jl)
      pltpu.sync_copy(rows_a, acc.at[rowv.at[jl]], add=True)
      return 0

    lax.fori_loop(0, NB, block_body, 0)
    plsc.subcore_barrier()

    @pl.when(s < NZ)
    def _dump():
      pltpu.sync_copy(acc.at[pl.ds(s * ZROWS, ZROWS)],
                      out_hbm.at[pl.ds(c * N + s * ZROWS, ZROWS)])

  return agg_kernel(x, row2, col2, val2, zrows_hbm)


def _tc_tail(partials, wt, bias):
  BLK = 1000
  grid = (N // BLK,)

  def tail_kernel(p0_ref, p1_ref, wt_ref, b_ref, mu_ref, lv_ref):
    agg = p0_ref[...] + p1_ref[...]
    y = jnp.dot(agg, wt_ref[...], preferred_element_type=jnp.float32)
    y = y + b_ref[...]
    mu_ref[...] = y[:, :D_LAT]
    lv_ref[...] = jnp.clip(y[:, D_LAT:], -10.0, 3.0)

  return pl.pallas_call(
      tail_kernel,
      grid=grid,
      in_specs=[
          pl.BlockSpec((BLK, D_IN), lambda i: (i, 0)),
          pl.BlockSpec((BLK, D_IN), lambda i: (i + N // BLK, 0)),
          pl.BlockSpec((D_IN, 2 * D_LAT), lambda i: (0, 0)),
          pl.BlockSpec((1, 2 * D_LAT), lambda i: (0, 0)),
      ],
      out_specs=[
          pl.BlockSpec((BLK, D_LAT), lambda i: (i, 0)),
          pl.BlockSpec((BLK, D_LAT), lambda i: (i, 0)),
      ],
      out_shape=[
          jax.ShapeDtypeStruct((N, D_LAT), jnp.float32),
          jax.ShapeDtypeStruct((N, D_LAT), jnp.float32),
      ],
  )(partials, partials, wt, bias)


@jax.jit
def kernel(x, adj_indices, adj_values, W_mu, b_mu, W_lv, b_lv):
  row2 = adj_indices[0].astype(jnp.int32).reshape(NW, NB, CB, K)
  col2 = adj_indices[1].astype(jnp.int32).reshape(NW, NB, CB, K)
  val2 = adj_values.reshape(NW, NB, CB, K)

  zrows = jnp.zeros((ZROWS, D_IN), jnp.float32)
  partials = _sc_aggregate(x, row2, col2, val2, zrows)

  wt = jnp.concatenate([W_mu, W_lv], axis=0).T
  bias = jnp.concatenate([b_mu, b_lv]).reshape(1, 2 * D_LAT)
  mu, logvar = _tc_tail(partials, wt, bias)
  return (mu, logvar)
